# baseline (speedup 1.0000x reference)
import functools

import jax
import jax.numpy as jnp
from jax import lax
from jax.experimental import pallas as pl
from jax.experimental.pallas import tpu as pltpu
from jax.experimental.pallas import tpu_sc as plsc

_D = 256
_BM = 512
_BN = 2048
_BL = 512


def _argmin_body(x_ref, xsq_ref, e_ref, idx_ref, runmin, runarg):
    j = pl.program_id(1)

    @pl.when(j == 0)
    def _():
        runmin[...] = jnp.full_like(runmin[...], jnp.inf)
        runarg[...] = jnp.zeros_like(runarg[...])

    x = x_ref[...]
    e = e_ref[...]
    xsq = xsq_ref[...][:, None]
    esq = jnp.sum(e * e, axis=1)
    mm = lax.dot_general(x, e, (((1,), (1,)), ((), ())),
                         preferred_element_type=jnp.float32)
    d2 = (xsq + esq[None, :]) - 2.0 * mm
    dist = jnp.sqrt(jnp.maximum(d2, 0.0))
    bmin = jnp.min(dist, axis=1, keepdims=True)
    ii = lax.broadcasted_iota(jnp.int32, dist.shape, 1) + j * _BN
    barg = jnp.min(jnp.where(dist == bmin, ii, jnp.int32(2**30)), axis=1)
    bminv = bmin[:, 0]
    better = bminv < runmin[...]
    runmin[...] = jnp.where(better, bminv, runmin[...])
    runarg[...] = jnp.where(better, barg, runarg[...])

    @pl.when(j == pl.num_programs(1) - 1)
    def _():
        idx_ref[...] = runarg[...]


def _argmin_call(xf, xsq, emb):
    m, d = xf.shape
    n = emb.shape[0]
    return pl.pallas_call(
        _argmin_body,
        grid=(m // _BM, n // _BN),
        in_specs=[
            pl.BlockSpec((_BM, d), lambda i, j: (i, 0)),
            pl.BlockSpec((_BM,), lambda i, j: (i,)),
            pl.BlockSpec((_BN, d), lambda i, j: (j, 0)),
        ],
        out_specs=pl.BlockSpec((_BM,), lambda i, j: (i,)),
        out_shape=jax.ShapeDtypeStruct((m,), jnp.int32),
        scratch_shapes=[
            pltpu.VMEM((_BM,), jnp.float32),
            pltpu.VMEM((_BM,), jnp.int32),
        ],
        compiler_params=pltpu.CompilerParams(
            dimension_semantics=("arbitrary", "arbitrary")),
    )(xf, xsq, emb)


def _sc_gather(table, idx):
    m = idx.shape[0]
    d = table.shape[1]
    nc, ns = 2, 16
    bpw = m // (nc * ns)
    mesh = plsc.VectorSubcoreMesh(core_axis_name="c", subcore_axis_name="s")

    @functools.partial(
        pl.kernel,
        mesh=mesh,
        out_type=jax.ShapeDtypeStruct((m, d), jnp.float32),
        scratch_types=[
            pltpu.VMEM((bpw,), jnp.int32),
            pltpu.VMEM((bpw, d), jnp.float32),
            pltpu.SemaphoreType.DMA,
        ],
    )
    def gather_k(table_hbm, idx_hbm, out_hbm, idx_v, rows_v, sem):
        wid = lax.axis_index("s") * nc + lax.axis_index("c")
        base = wid * bpw
        pltpu.sync_copy(idx_hbm.at[pl.ds(base, bpw)], idx_v)
        pltpu.async_copy(table_hbm.at[idx_v], rows_v, sem).wait()
        pltpu.sync_copy(rows_v, out_hbm.at[pl.ds(base, bpw)])

    return gather_k(table, idx)


def _loss_body(x_ref, q_ref, qout_ref, sum_ref, acc):
    i = pl.program_id(0)

    @pl.when(i == 0)
    def _():
        acc[0] = 0.0

    xv = x_ref[...]
    qv = q_ref[...]
    diff = qv - xv
    qout_ref[...] = xv + diff
    acc[0] += jnp.sum(diff * diff)

    @pl.when(i == pl.num_programs(0) - 1)
    def _():
        sum_ref[0] = acc[0]


def _loss_call(xf, q):
    m, d = xf.shape
    qout, ssum = pl.pallas_call(
        _loss_body,
        grid=(m // _BL,),
        in_specs=[
            pl.BlockSpec((_BL, d), lambda i: (i, 0)),
            pl.BlockSpec((_BL, d), lambda i: (i, 0)),
        ],
        out_specs=[
            pl.BlockSpec((_BL, d), lambda i: (i, 0)),
            pl.BlockSpec(memory_space=pltpu.SMEM),
        ],
        out_shape=[
            jax.ShapeDtypeStruct((m, d), jnp.float32),
            jax.ShapeDtypeStruct((1,), jnp.float32),
        ],
        scratch_shapes=[pltpu.SMEM((1,), jnp.float32)],
        compiler_params=pltpu.CompilerParams(
            dimension_semantics=("arbitrary",)),
    )(xf, q)
    return qout, ssum


def kernel(x, embeddings):
    orig_shape = x.shape
    d = orig_shape[-1]
    xf = x.reshape(-1, d)
    xsq = jnp.sum(xf * xf, axis=1)
    idx = _argmin_call(xf, xsq, embeddings)
    q = _sc_gather(embeddings, idx)
    qout, ssum = _loss_call(xf, q)
    mean_sq = ssum[0] / jnp.float32(xf.size)
    commitment_loss = mean_sq * jnp.float32(0.25)
    codebook_loss = mean_sq
    total = commitment_loss + codebook_loss
    return (qout.reshape(orig_shape),
            idx.reshape(orig_shape[:-1]),
            commitment_loss,
            codebook_loss,
            total)
